# baseline (speedup 1.0000x reference)
import jax
import jax.numpy as jnp
from jax.experimental import pallas as pl
from jax.experimental.pallas import tpu as pltpu

_BS, _C, _N, _F = 128, 32, 512, 16
_MASK_RATIO = 0.4
_LEN_KEEP = int(_N * (1 - _MASK_RATIO))
_ROWS = _BS * _C
_ROW_BLOCK = 128
_CHUNK = 128


def _body(noise_ref, patch_ref, out_ref, mask_ref):
    r = noise_ref.shape[0]
    n = noise_ref.shape[1]
    k = _LEN_KEEP
    bits = jax.lax.bitcast_convert_type(noise_ref[...], jnp.int32)

    lo = jnp.zeros((r, 1), jnp.int32)
    hi = jnp.full((r, 1), jnp.int32(0x3F800000))

    def p1(_, lohi):
        lo, hi = lohi
        mid = lo + (hi - lo) // 2
        cnt = jnp.sum((bits <= mid).astype(jnp.int32), axis=1, keepdims=True)
        ge = cnt >= k
        return jnp.where(ge, lo, mid + 1), jnp.where(ge, mid, hi)

    lo, hi = jax.lax.fori_loop(0, 30, p1, (lo, hi))
    v = lo

    cl = jnp.sum((bits < v).astype(jnp.int32), axis=1, keepdims=True)
    need = k - cl
    idx = jax.lax.broadcasted_iota(jnp.int32, (r, n), 1)
    eq = bits == v
    lo2 = jnp.zeros((r, 1), jnp.int32)
    hi2 = jnp.full((r, 1), jnp.int32(n - 1))

    def p2(_, lohi):
        lo2, hi2 = lohi
        mid = lo2 + (hi2 - lo2) // 2
        cnt = jnp.sum((eq & (idx <= mid)).astype(jnp.int32), axis=1,
                      keepdims=True)
        ge = cnt >= need
        return jnp.where(ge, lo2, mid + 1), jnp.where(ge, mid, hi2)

    lo2, hi2 = jax.lax.fori_loop(0, 9, p2, (lo2, hi2))
    t = lo2

    keep = (bits < v) | (eq & (idx <= t))
    masked = ~keep
    mask_ref[...] = masked

    ci = jax.lax.broadcasted_iota(jnp.int32, (_CHUNK, _CHUNK * _F), 1) // _F
    ri = jax.lax.broadcasted_iota(jnp.int32, (_CHUNK, _CHUNK * _F), 0)
    e = (ci == ri).astype(jnp.bfloat16)

    mbf = masked.astype(jnp.bfloat16)
    for j in range(n // _CHUNK):
        mf = jax.lax.dot(mbf[:, j * _CHUNK:(j + 1) * _CHUNK], e,
                         preferred_element_type=jnp.float32)
        c0 = j * _CHUNK * _F
        c1 = (j + 1) * _CHUNK * _F
        out_ref[:, c0:c1] = jnp.where(mf > 0.5, jnp.float32(0.0),
                                      patch_ref[:, c0:c1])


@jax.jit
def kernel(patch_input, noise):
    bs, c, n, f = patch_input.shape
    rows = bs * c
    patch2 = patch_input.reshape(rows, n * f)
    noise2 = noise.reshape(rows, n)
    grid = (rows // _ROW_BLOCK,)
    out, mask = pl.pallas_call(
        _body,
        grid=grid,
        in_specs=[
            pl.BlockSpec((_ROW_BLOCK, n), lambda i: (i, 0)),
            pl.BlockSpec((_ROW_BLOCK, n * f), lambda i: (i, 0)),
        ],
        out_specs=[
            pl.BlockSpec((_ROW_BLOCK, n * f), lambda i: (i, 0)),
            pl.BlockSpec((_ROW_BLOCK, n), lambda i: (i, 0)),
        ],
        out_shape=[
            jax.ShapeDtypeStruct((rows, n * f), jnp.float32),
            jax.ShapeDtypeStruct((rows, n), jnp.bool_),
        ],
    )(noise2, patch2)
    return out.reshape(bs, c, n, f), mask.reshape(bs, c, n)


# speedup vs baseline: 1.1666x; 1.1666x over previous
import jax
import jax.numpy as jnp
from jax.experimental import pallas as pl
from jax.experimental.pallas import tpu as pltpu

_BS, _C, _N, _F = 128, 32, 512, 16
_MASK_RATIO = 0.4
_LEN_KEEP = int(_N * (1 - _MASK_RATIO))


def _body(noise_ref, patch_ref, out_ref, mask_ref):
    c = noise_ref.shape[1]
    n = noise_ref.shape[2]
    k = _LEN_KEEP
    bits2 = jax.lax.bitcast_convert_type(noise_ref[0], jnp.int32)
    bits = jnp.transpose(bits2)

    lo = jnp.zeros((1, c), jnp.int32)
    hi = jnp.full((1, c), jnp.int32(0x3F800000))

    def p1(_, lohi):
        lo, hi = lohi
        mid = lo + (hi - lo) // 2
        cnt = jnp.sum((bits <= mid).astype(jnp.int32), axis=0, keepdims=True)
        ge = cnt >= k
        return jnp.where(ge, lo, mid + 1), jnp.where(ge, mid, hi)

    lo, hi = jax.lax.fori_loop(0, 30, p1, (lo, hi))
    v = lo

    cl = jnp.sum((bits < v).astype(jnp.int32), axis=0, keepdims=True)
    need = k - cl
    idx = jax.lax.broadcasted_iota(jnp.int32, (n, c), 0)
    eq = bits == v
    lo2 = jnp.zeros((1, c), jnp.int32)
    hi2 = jnp.full((1, c), jnp.int32(n - 1))

    def p2(_, lohi):
        lo2, hi2 = lohi
        mid = lo2 + (hi2 - lo2) // 2
        cnt = jnp.sum((eq & (idx <= mid)).astype(jnp.int32), axis=0,
                      keepdims=True)
        ge = cnt >= need
        return jnp.where(ge, lo2, mid + 1), jnp.where(ge, mid, hi2)

    lo2, hi2 = jax.lax.fori_loop(0, 9, p2, (lo2, hi2))
    t = lo2

    keep_t = (bits < v) | (eq & (idx <= t))
    masked_t = (~keep_t).astype(jnp.int32)
    mask_ref[...] = (jnp.transpose(masked_t) != 0)[None]

    for j in range(c):
        col = masked_t[:, j:j + 1] != 0
        m3 = jnp.broadcast_to(col, (n, _F))
        out_ref[0, j] = jnp.where(m3, jnp.float32(0.0), patch_ref[0, j])


@jax.jit
def kernel(patch_input, noise):
    bs, c, n, f = patch_input.shape
    grid = (bs,)
    out, mask = pl.pallas_call(
        _body,
        grid=grid,
        in_specs=[
            pl.BlockSpec((1, c, n), lambda i: (i, 0, 0)),
            pl.BlockSpec((1, c, n, f), lambda i: (i, 0, 0, 0)),
        ],
        out_specs=[
            pl.BlockSpec((1, c, n, f), lambda i: (i, 0, 0, 0)),
            pl.BlockSpec((1, c, n), lambda i: (i, 0, 0)),
        ],
        out_shape=[
            jax.ShapeDtypeStruct((bs, c, n, f), jnp.float32),
            jax.ShapeDtypeStruct((bs, c, n), jnp.bool_),
        ],
    )(noise, patch_input)
    return out, mask
